# baseline (speedup 1.0000x reference)
import functools

import jax
import jax.numpy as jnp
from jax import lax
from jax.experimental import pallas as pl
from jax.experimental.pallas import tpu as pltpu
from jax.experimental.pallas import tpu_sc as plsc

B = 16384
D = 64
LANES = 16
CHUNK = 128


def _make_sc_dots(nc: int, ns: int):
    nw = nc * ns
    b_per_w = B // nw
    nch = b_per_w // CHUNK
    groups_per_ch = CHUNK // LANES
    n_groups = b_per_w // LANES

    mesh = plsc.VectorSubcoreMesh(core_axis_name="c", subcore_axis_name="s")

    @functools.partial(
        pl.kernel,
        mesh=mesh,
        compiler_params=pltpu.CompilerParams(
            use_tc_tiling_on_sc=False, needs_layout_passes=False),
        out_type=jax.ShapeDtypeStruct((B,), jnp.float32),
        scratch_types=[
            pltpu.VMEM((nch, CHUNK), jnp.int32),
            pltpu.VMEM((nch, CHUNK), jnp.int32),
            pltpu.VMEM((nch, CHUNK, D), jnp.float32),
            pltpu.VMEM((nch, CHUNK, D), jnp.float32),
            pltpu.VMEM((b_per_w,), jnp.float32),
            pltpu.SemaphoreType.DMA,
        ],
    )
    def sc_dots(wi_hbm, wc_hbm, iw_hbm, cw_hbm, out_hbm,
                idx_a, idx_b, rows_a, rows_b, dots, sem):
        wid = lax.axis_index("s") * nc + lax.axis_index("c")
        pltpu.sync_copy(iw_hbm.at[wid], idx_a)
        pltpu.sync_copy(cw_hbm.at[wid], idx_b)
        copies = []
        for ch in range(nch):
            copies.append(pltpu.async_copy(wi_hbm.at[idx_a.at[ch]], rows_a.at[ch], sem))
            copies.append(pltpu.async_copy(wc_hbm.at[idx_b.at[ch]], rows_b.at[ch], sem))
        for c in copies:
            c.wait()

        lane = lax.iota(jnp.int32, LANES)

        def body(g, carry):
            ch = g // groups_per_ch
            row = (g % groups_per_ch) * LANES + lane
            chv = jnp.full((LANES,), 0, jnp.int32) + ch
            acc = jnp.zeros((LANES,), jnp.float32)
            for j in range(D):
                colv = jnp.full((LANES,), j, jnp.int32)
                va = plsc.load_gather(rows_a, [chv, row, colv])
                vb = plsc.load_gather(rows_b, [chv, row, colv])
                acc = acc + va * vb
            dots[pl.ds(g * LANES, LANES)] = acc
            return carry

        lax.fori_loop(0, n_groups, body, 0)
        pltpu.sync_copy(dots, out_hbm.at[pl.ds(wid * b_per_w, b_per_w)])

    return sc_dots


def _loss_body(x_ref, o_ref):
    x = x_ref[...]
    ls = jnp.minimum(x, 0.0) - jnp.log1p(jnp.exp(-jnp.abs(x)))
    o_ref[0, 0] = -jnp.sum(ls) * (1.0 / B)


def kernel(W_input, W_context, input_word, context_word):
    info = plsc.get_sparse_core_info()
    nc, ns = info.num_cores, info.num_subcores
    nw = nc * ns
    b_per_w = B // nw
    nch = b_per_w // CHUNK

    iw = input_word.astype(jnp.int32).reshape(nw, nch, CHUNK)
    cw = context_word.astype(jnp.int32).reshape(nw, nch, CHUNK)

    dots = _make_sc_dots(nc, ns)(W_input, W_context, iw, cw)

    loss = pl.pallas_call(
        _loss_body,
        out_shape=jax.ShapeDtypeStruct((1, 1), jnp.float32),
        out_specs=pl.BlockSpec(memory_space=pltpu.SMEM),
    )(dots.reshape(B // 128, 128))
    return loss.reshape(())


# speedup vs baseline: 1.5264x; 1.5264x over previous
import functools

import jax
import jax.numpy as jnp
from jax import lax
from jax.experimental import pallas as pl
from jax.experimental.pallas import tpu as pltpu
from jax.experimental.pallas import tpu_sc as plsc

B = 16384
D = 64
LANES = 16
BP = 256


def _make_sc_dots(nc: int, ns: int):
    nw = nc * ns
    b_per_w = B // nw
    n_passes = b_per_w // BP
    gpp = BP // LANES

    mesh = plsc.VectorSubcoreMesh(core_axis_name="c", subcore_axis_name="s")

    @functools.partial(
        pl.kernel,
        mesh=mesh,
        compiler_params=pltpu.CompilerParams(
            use_tc_tiling_on_sc=True, needs_layout_passes=False),
        out_type=jax.ShapeDtypeStruct((B,), jnp.float32),
        scratch_types=[
            pltpu.VMEM((b_per_w,), jnp.int32),
            pltpu.VMEM((b_per_w,), jnp.int32),
            pltpu.VMEM((2 * LANES, D), jnp.float32),
            pltpu.VMEM((2 * LANES, D), jnp.float32),
            pltpu.VMEM((BP, 128), jnp.float32),
            pltpu.VMEM((BP, 128), jnp.float32),
            pltpu.VMEM((b_per_w,), jnp.float32),
            pltpu.SemaphoreType.DMA,
            pltpu.SemaphoreType.DMA,
        ],
    )
    def sc_dots(wi_hbm, wc_hbm, iw_hbm, cw_hbm, dummy_hbm, out_hbm,
                idx_a, idx_b, stage_a, stage_b, rows_a, rows_b, dots,
                sem0, sem1):
        wid = lax.axis_index("s") * nc + lax.axis_index("c")
        base = wid * b_per_w
        pltpu.sync_copy(iw_hbm.at[pl.ds(base, b_per_w)], idx_a)
        pltpu.sync_copy(cw_hbm.at[pl.ds(base, b_per_w)], idx_b)

        lane = lax.iota(jnp.int32, LANES)

        def fire(pb, u, pp, sem):
            va = idx_a[pl.ds(pb + u * LANES, LANES)]
            vb = idx_b[pl.ds(pb + u * LANES, LANES)]
            soff = pp * LANES
            for l in range(LANES):
                ra = lax.reduce_sum_p.bind(
                    jnp.where(lane == l, va, 0), axes=(0,))
                rb = lax.reduce_sum_p.bind(
                    jnp.where(lane == l, vb, 0), axes=(0,))
                pltpu.async_copy(
                    wi_hbm.at[pl.ds(ra, 1), :],
                    stage_a.at[pl.ds(soff + l, 1), :], sem)
                pltpu.async_copy(
                    wc_hbm.at[pl.ds(rb, 1), :],
                    stage_b.at[pl.ds(soff + l, 1), :], sem)

        def drain_and_copy(u, pp, sem):
            soff = pp * LANES
            pltpu.make_async_copy(
                dummy_hbm, stage_a.at[pl.ds(soff, LANES), :], sem).wait()
            pltpu.make_async_copy(
                dummy_hbm, stage_b.at[pl.ds(soff, LANES), :], sem).wait()
            for l in range(LANES):
                dst_a = rows_a.at[u * LANES + l]
                dst_b = rows_b.at[u * LANES + l]
                for c in range(D // LANES):
                    sl = pl.ds(c * LANES, LANES)
                    dst_a[sl] = stage_a[soff + l, sl]
                    dst_b[sl] = stage_b[soff + l, sl]

        def run_pass(p, carry):
            pb = p * BP

            def pipe(u, c):
                @pl.when(u < gpp)
                def _():
                    @pl.when(u % 2 == 0)
                    def _():
                        fire(pb, u, 0, sem0)

                    @pl.when(u % 2 == 1)
                    def _():
                        fire(pb, u, 1, sem1)

                @pl.when(u > 0)
                def _():
                    @pl.when((u - 1) % 2 == 0)
                    def _():
                        drain_and_copy(u - 1, 0, sem0)

                    @pl.when((u - 1) % 2 == 1)
                    def _():
                        drain_and_copy(u - 1, 1, sem1)

                return c

            lax.fori_loop(0, gpp + 1, pipe, 0)

            def compute(g, c):
                kv = g * LANES + lane
                acc = jnp.zeros((LANES,), jnp.float32)
                for j in range(D):
                    jv = jnp.full((LANES,), j, jnp.int32)
                    va = plsc.load_gather(rows_a, [kv, jv])
                    vb = plsc.load_gather(rows_b, [kv, jv])
                    acc = acc + va * vb
                dots[pl.ds(pb + g * LANES, LANES)] = acc
                return c

            lax.fori_loop(0, gpp, compute, 0)
            return carry

        lax.fori_loop(0, n_passes, run_pass, 0)
        pltpu.sync_copy(dots, out_hbm.at[pl.ds(base, b_per_w)])

    return sc_dots


def _loss_body(x_ref, o_ref):
    x = x_ref[...]
    ls = jnp.minimum(x, 0.0) - jnp.log1p(jnp.exp(-jnp.abs(x)))
    o_ref[0, 0] = -jnp.sum(ls) * (1.0 / B)


def kernel(W_input, W_context, input_word, context_word):
    info = plsc.get_sparse_core_info()
    nc, ns = info.num_cores, info.num_subcores

    iw = input_word.astype(jnp.int32)
    cw = context_word.astype(jnp.int32)
    dummy = jnp.zeros((LANES, D), jnp.float32)

    dots = _make_sc_dots(nc, ns)(W_input, W_context, iw, cw, dummy)

    loss = pl.pallas_call(
        _loss_body,
        out_shape=jax.ShapeDtypeStruct((1, 1), jnp.float32),
        out_specs=pl.BlockSpec(memory_space=pltpu.SMEM),
    )(dots.reshape(B // 128, 128))
    return loss.reshape(())
